# Initial kernel scaffold; baseline (speedup 1.0000x reference)
import jax
import jax.numpy as jnp
from jax.experimental import pallas as pl


def kernel(pos_u, pos_v, neg_v, u_weight, v_weight):
    raise NotImplementedError("write your pallas kernel here")


# speedup vs baseline: 4.7710x; 4.7710x over previous
import functools

import jax
import jax.numpy as jnp
from jax import lax
from jax.experimental import pallas as pl
from jax.experimental.pallas import tpu as pltpu
from jax.experimental.pallas import tpu_sc as plsc

EMB_DIM = 64
NEG = 20
T = NEG + 1
LANES = 16
NC, NS = 2, 16
NW = NC * NS
C = 64
GB = 128


def _sc_scores(pos_u, tgt_idx, u_weight, v_weight):
    B = pos_u.shape[0]
    PW = B // NW
    NCHUNK = PW // C

    mesh = plsc.VectorSubcoreMesh(core_axis_name="c", subcore_axis_name="s")

    @functools.partial(
        pl.kernel,
        out_type=jax.ShapeDtypeStruct((B * T,), jnp.float32),
        mesh=mesh,
        scratch_types=[
            pltpu.VMEM((PW,), jnp.int32),
            pltpu.VMEM((PW * T,), jnp.int32),
            pltpu.VMEM((C, EMB_DIM), jnp.float32),
            pltpu.VMEM((C * T, EMB_DIM), jnp.float32),
            pltpu.VMEM((C * T,), jnp.float32),
            pltpu.SemaphoreType.DMA,
            pltpu.SemaphoreType.DMA,
        ],
        compiler_params=pltpu.CompilerParams(
            needs_layout_passes=False, use_tc_tiling_on_sc=False),
    )
    def k(pu_hbm, tgt_hbm, uw_hbm, vw_hbm, out_hbm,
          uidx_v, tidx_v, urows_v, trows_v, sc_v, sem_u, sem_t):
        wid = lax.axis_index("s") * NC + lax.axis_index("c")
        base = wid * PW
        pltpu.sync_copy(pu_hbm.at[pl.ds(base, PW)], uidx_v)
        pltpu.sync_copy(tgt_hbm.at[pl.ds(base * T, PW * T)], tidx_v)

        lane = lax.iota(jnp.int32, 16)
        last = lane == (LANES - 1)

        def chunk_body(cix, carry):
            coff = cix * C
            cu = pltpu.async_copy(
                uw_hbm.at[uidx_v.at[pl.ds(coff, C)]], urows_v, sem_u)
            nfull, rem = (C * T) // GB, (C * T) % GB
            copies = []
            for b in range(nfull):
                copies.append(pltpu.async_copy(
                    vw_hbm.at[tidx_v.at[pl.ds(coff * T + b * GB, GB)]],
                    trows_v.at[pl.ds(b * GB, GB)], sem_t))
            if rem:
                copies.append(pltpu.async_copy(
                    vw_hbm.at[tidx_v.at[pl.ds(coff * T + nfull * GB, rem)]],
                    trows_v.at[pl.ds(nfull * GB, rem)], sem_t))
            cu.wait()
            for cp in copies:
                cp.wait()

            def pair_body(i, carry2):
                u0 = urows_v[i, pl.ds(0, 16)]
                u1 = urows_v[i, pl.ds(16, 16)]
                u2 = urows_v[i, pl.ds(32, 16)]
                u3 = urows_v[i, pl.ds(48, 16)]
                for t in range(T):
                    r = i * T + t
                    p = (u0 * trows_v[r, pl.ds(0, 16)]
                         + u1 * trows_v[r, pl.ds(16, 16)]
                         + u2 * trows_v[r, pl.ds(32, 16)]
                         + u3 * trows_v[r, pl.ds(48, 16)])
                    cum = plsc.cumsum(p)
                    plsc.store_scatter(
                        sc_v, [jnp.full((16,), r, jnp.int32)], cum, mask=last)
                return carry2

            lax.fori_loop(0, C, pair_body, 0, unroll=False)
            pltpu.sync_copy(sc_v, out_hbm.at[pl.ds((base + coff) * T, C * T)])
            return carry

        lax.fori_loop(0, NCHUNK, chunk_body, 0, unroll=False)

    return k(pos_u, tgt_idx, u_weight, v_weight)


def _tc_loss(scores):
    B = scores.shape[0]
    blk = 2048
    grid = B // blk

    def body(s_ref, pos_ref, neg_ref):
        g = pl.program_id(0)
        x = s_ref[...]
        xc = jnp.clip(x, -6.0, 6.0)
        col = lax.broadcasted_iota(jnp.int32, x.shape, 1)
        ispos = col == 0
        isneg = (col >= 1) & (col < T)
        elem = jnp.log1p(jnp.exp(jnp.where(ispos, -xc, xc)))
        pos_p = jnp.sum(jnp.where(ispos, elem, 0.0))
        neg_p = jnp.sum(jnp.where(isneg, elem, 0.0))

        @pl.when(g == 0)
        def _():
            pos_ref[...] = jnp.zeros((1, 1), jnp.float32)
            neg_ref[...] = jnp.zeros((1, 1), jnp.float32)

        pos_ref[...] += jnp.full((1, 1), pos_p, jnp.float32)
        neg_ref[...] += jnp.full((1, 1), neg_p, jnp.float32)

    pos, neg = pl.pallas_call(
        body,
        grid=(grid,),
        in_specs=[pl.BlockSpec((blk, T), lambda g: (g, 0))],
        out_specs=[pl.BlockSpec((1, 1), lambda g: (0, 0)),
                   pl.BlockSpec((1, 1), lambda g: (0, 0))],
        out_shape=[jax.ShapeDtypeStruct((1, 1), jnp.float32)] * 2,
    )(scores)
    return pos[0, 0], neg[0, 0]


@jax.jit
def kernel(pos_u, pos_v, neg_v, u_weight, v_weight):
    B = pos_u.shape[0]
    tgt_idx = jnp.concatenate([pos_v[:, None], neg_v], axis=1).reshape(B * T)
    scores = _sc_scores(pos_u, tgt_idx, u_weight, v_weight)
    return _tc_loss(scores.reshape(B, T))


# speedup vs baseline: 5.8599x; 1.2282x over previous
import functools

import jax
import jax.numpy as jnp
from jax import lax
from jax.experimental import pallas as pl
from jax.experimental.pallas import tpu as pltpu
from jax.experimental.pallas import tpu_sc as plsc

EMB_DIM = 64
NEG = 20
T = NEG + 1
LANES = 16
NC, NS = 2, 16
NW = NC * NS
C = 16
GB = 128


def _compact(wt):
    rows = wt.shape[1]
    blk = 8192
    grid = pl.cdiv(rows, blk)

    def body(in_ref, out_ref):
        x = in_ref[...]
        y = x.T.reshape(blk // 2, 2, EMB_DIM)
        out_ref[:, 0:EMB_DIM] = y[:, 0, :]
        out_ref[:, EMB_DIM:2 * EMB_DIM] = y[:, 1, :]

    return pl.pallas_call(
        body,
        grid=(grid,),
        in_specs=[pl.BlockSpec((EMB_DIM, blk), lambda g: (0, g))],
        out_specs=pl.BlockSpec((blk // 2, 2 * EMB_DIM), lambda g: (g, 0)),
        out_shape=jax.ShapeDtypeStruct((rows // 2, 2 * EMB_DIM), jnp.float32),
    )(wt)


def _sc_scores(pu_raw, pu_pair, t_raw, t_pair, uc, vc, B):
    PW = B // NW
    NCHUNK = PW // C
    CT = C * T

    mesh = plsc.VectorSubcoreMesh(core_axis_name="c", subcore_axis_name="s")

    @functools.partial(
        pl.kernel,
        out_type=jax.ShapeDtypeStruct((B * T,), jnp.float32),
        mesh=mesh,
        scratch_types=[
            pltpu.VMEM((PW,), jnp.int32),
            pltpu.VMEM((PW,), jnp.int32),
            pltpu.VMEM((PW * T,), jnp.int32),
            pltpu.VMEM((PW * T,), jnp.int32),
            pltpu.VMEM((2, C, 2 * EMB_DIM), jnp.float32),
            pltpu.VMEM((2, CT, 2 * EMB_DIM), jnp.float32),
            pltpu.VMEM((CT,), jnp.float32),
            pltpu.SemaphoreType.DMA,
            pltpu.SemaphoreType.DMA,
        ],
        compiler_params=pltpu.CompilerParams(
            needs_layout_passes=False, use_tc_tiling_on_sc=True),
    )
    def k(puraw_hbm, pupair_hbm, traw_hbm, tpair_hbm, uc_hbm, vc_hbm, out_hbm,
          uraw_v, upair_v, traw_v, tpair_v, ubuf_v, tbuf_v, sc_v, sem_u, sem_t):
        wid = lax.axis_index("s") * NC + lax.axis_index("c")
        base = wid * PW
        pltpu.sync_copy(puraw_hbm.at[pl.ds(base, PW)], uraw_v)
        pltpu.sync_copy(pupair_hbm.at[pl.ds(base, PW)], upair_v)
        pltpu.sync_copy(traw_hbm.at[pl.ds(base * T, PW * T)], traw_v)
        pltpu.sync_copy(tpair_hbm.at[pl.ds(base * T, PW * T)], tpair_v)

        lane = lax.iota(jnp.int32, 16)
        last = lane == (LANES - 1)
        nfull, rem = CT // GB, CT % GB

        def t_copies(cix, bank):
            cs = []
            for b in range(nfull):
                cs.append(pltpu.make_async_copy(
                    vc_hbm.at[tpair_v.at[pl.ds(cix * CT + b * GB, GB)]],
                    tbuf_v.at[bank, pl.ds(b * GB, GB)], sem_t))
            if rem:
                cs.append(pltpu.make_async_copy(
                    vc_hbm.at[tpair_v.at[pl.ds(cix * CT + nfull * GB, rem)]],
                    tbuf_v.at[bank, pl.ds(nfull * GB, rem)], sem_t))
            return cs

        def u_copy(cix, bank):
            return pltpu.make_async_copy(
                uc_hbm.at[upair_v.at[pl.ds(cix * C, C)]],
                ubuf_v.at[bank], sem_u)

        for cp in t_copies(0, 0):
            cp.start()
        u_copy(0, 0).start()

        def chunk_body(cix, carry):
            bank = lax.rem(cix, 2)
            nbank = 1 - bank

            @pl.when(cix < NCHUNK - 1)
            def _():
                for cp in t_copies(cix + 1, nbank):
                    cp.start()
                u_copy(cix + 1, nbank).start()

            for cp in t_copies(cix, bank):
                cp.wait()
            u_copy(cix, bank).wait()

            def pair_body(p, carry2):
                gp = cix * C + p
                uraw = plsc.load_gather(uraw_v, [jnp.full((16,), gp, jnp.int32)])
                ucol = (uraw & 1) * EMB_DIM + lane
                pv = jnp.full((16,), p, jnp.int32)
                u0 = plsc.load_gather(ubuf_v.at[bank], [pv, ucol])
                u1 = plsc.load_gather(ubuf_v.at[bank], [pv, ucol + 16])
                u2 = plsc.load_gather(ubuf_v.at[bank], [pv, ucol + 32])
                u3 = plsc.load_gather(ubuf_v.at[bank], [pv, ucol + 48])
                for t in range(T):
                    q = p * T + t
                    traw = plsc.load_gather(
                        traw_v, [jnp.full((16,), cix * CT + q, jnp.int32)])
                    tcol = (traw & 1) * EMB_DIM + lane
                    qv = jnp.full((16,), q, jnp.int32)
                    t0 = plsc.load_gather(tbuf_v.at[bank], [qv, tcol])
                    t1 = plsc.load_gather(tbuf_v.at[bank], [qv, tcol + 16])
                    t2 = plsc.load_gather(tbuf_v.at[bank], [qv, tcol + 32])
                    t3 = plsc.load_gather(tbuf_v.at[bank], [qv, tcol + 48])
                    part = u0 * t0 + u1 * t1 + u2 * t2 + u3 * t3
                    cum = plsc.cumsum(part)
                    plsc.store_scatter(sc_v, [qv], cum, mask=last)
                return carry2

            lax.fori_loop(0, C, pair_body, 0, unroll=False)
            pltpu.sync_copy(
                sc_v, out_hbm.at[pl.ds((base + cix * C) * T, CT)])
            return carry

        lax.fori_loop(0, NCHUNK, chunk_body, 0, unroll=False)

    return k(pu_raw, pu_pair, t_raw, t_pair, uc, vc)


def _tc_loss(scores):
    B = scores.shape[0]
    blk = 2048
    grid = B // blk

    def body(s_ref, pos_ref, neg_ref):
        g = pl.program_id(0)
        x = s_ref[...]
        xc = jnp.clip(x, -6.0, 6.0)
        col = lax.broadcasted_iota(jnp.int32, x.shape, 1)
        ispos = col == 0
        isneg = (col >= 1) & (col < T)
        elem = jnp.log1p(jnp.exp(jnp.where(ispos, -xc, xc)))
        pos_p = jnp.sum(jnp.where(ispos, elem, 0.0))
        neg_p = jnp.sum(jnp.where(isneg, elem, 0.0))

        @pl.when(g == 0)
        def _():
            pos_ref[...] = jnp.zeros((1, 1), jnp.float32)
            neg_ref[...] = jnp.zeros((1, 1), jnp.float32)

        pos_ref[...] += jnp.full((1, 1), pos_p, jnp.float32)
        neg_ref[...] += jnp.full((1, 1), neg_p, jnp.float32)

    pos, neg = pl.pallas_call(
        body,
        grid=(grid,),
        in_specs=[pl.BlockSpec((blk, T), lambda g: (g, 0))],
        out_specs=[pl.BlockSpec((1, 1), lambda g: (0, 0)),
                   pl.BlockSpec((1, 1), lambda g: (0, 0))],
        out_shape=[jax.ShapeDtypeStruct((1, 1), jnp.float32)] * 2,
    )(scores)
    return pos[0, 0], neg[0, 0]


@jax.jit
def kernel(pos_u, pos_v, neg_v, u_weight, v_weight):
    B = pos_u.shape[0]
    tgt = jnp.concatenate([pos_v[:, None], neg_v], axis=1).reshape(B * T)
    uc = _compact(u_weight.T)
    vc = _compact(v_weight.T)
    scores = _sc_scores(pos_u, pos_u >> 1, tgt, tgt >> 1, uc, vc, B)
    return _tc_loss(scores.reshape(B, T))
